# baseline (speedup 1.0000x reference)
import functools

import jax
import jax.numpy as jnp
from jax import lax
from jax.experimental import pallas as pl
from jax.experimental.pallas import tpu as pltpu
from jax.experimental.pallas import tpu_sc as plsc

HIDDEN = 128
MAX_NB = 6
LANES = 16
NW = 32
NGRP = HIDDEN // LANES


def _mesh():
    return plsc.VectorSubcoreMesh(core_axis_name="c", subcore_axis_name="s")


def _worker_id():
    return lax.axis_index("s") * 2 + lax.axis_index("c")


def _my_range(nchunk):
    wid = _worker_id()
    lo = nchunk // NW
    extra = nchunk - lo * NW
    n_my = lo + (wid < extra)
    base = wid * lo + jnp.minimum(wid, extra)
    return base, n_my


def _sc_gather_rows(table, idx3):
    nchunk = idx3.shape[0]
    chunk = 128

    @functools.partial(
        pl.kernel, mesh=_mesh(),
        out_type=jax.ShapeDtypeStruct((nchunk * chunk, HIDDEN), jnp.float32),
        scratch_types=[
            pltpu.VMEM((2, 1, chunk), jnp.int32),
            pltpu.VMEM((2, chunk, HIDDEN), jnp.float32),
            pltpu.SemaphoreType.DMA((2,)),
            pltpu.SemaphoreType.DMA((2,)),
        ],
    )
    def k(table_hbm, idx_hbm, out_hbm, idx_v, rows_v, gsem, osem):
        base, n_my = _my_range(nchunk)

        def gcp(slot):
            return pltpu.make_async_copy(
                table_hbm.at[idx_v.at[slot, 0]],
                rows_v.at[slot], gsem.at[slot])

        def ocp(slot, c):
            return pltpu.make_async_copy(
                rows_v.at[slot], out_hbm.at[pl.ds(c * chunk, chunk)],
                osem.at[slot])

        def issue(slot, i):
            c = base + i
            pltpu.sync_copy(idx_hbm.at[c], idx_v.at[slot])

            @pl.when(i >= 2)
            def _():
                ocp(slot, c - 2).wait()

            gcp(slot).start()

        def finish(slot, i):
            c = base + i
            gcp(slot).wait()
            ocp(slot, c).start()

        issue(0, 0)

        def pair(p, carry):
            i0 = 2 * p
            issue(1, i0 + 1)
            finish(0, i0)

            @pl.when(i0 + 2 < n_my)
            def _():
                issue(0, i0 + 2)

            finish(1, i0 + 1)
            return carry

        lax.fori_loop(0, n_my // 2, pair, 0)

        @pl.when(n_my % 2 == 1)
        def _():
            finish(0, n_my - 1)

        @pl.when(n_my % 2 == 0)
        def _():
            ocp(0, base + n_my - 2).wait()
            ocp(1, base + n_my - 1).wait()

        @pl.when(n_my % 2 == 1)
        def _():
            ocp(0, base + n_my - 1).wait()

            @pl.when(n_my >= 2)
            def _():
                ocp(1, base + n_my - 2).wait()

    return k(table, idx3)


def _decode_lo(w):
    return lax.bitcast_convert_type(lax.shift_left(w, 16), jnp.float32)


def _decode_hi(w):
    return lax.bitcast_convert_type(
        lax.bitwise_and(w, jnp.int32(-65536)), jnp.float32)


def _sc_msg(hw, lin, idx3, gather_packed):
    e_rows = lin.shape[0]
    chunk = 64
    nidx = chunk * MAX_NB // 128
    nchunk = e_rows // chunk
    gdtype = jnp.int32 if gather_packed else jnp.float32

    @functools.partial(
        pl.kernel, mesh=_mesh(),
        out_type=jax.ShapeDtypeStruct((e_rows, HIDDEN), jnp.int32),
        scratch_types=[
            pltpu.VMEM((2, nidx, 128), jnp.int32),
            pltpu.VMEM((2, chunk * MAX_NB, HIDDEN), gdtype),
            pltpu.VMEM((2, chunk, HIDDEN), jnp.int32),
            pltpu.SemaphoreType.DMA((2,)),
            pltpu.SemaphoreType.DMA((2,)),
            pltpu.SemaphoreType.DMA((2,)),
        ],
    )
    def k(hw_hbm, skip_hbm, idx_hbm, out_hbm, idx_v, rows_v, io_v, gsem,
          hsem, osem):
        base, n_my = _my_range(nchunk)

        def gcp(slot, j):
            return pltpu.make_async_copy(
                hw_hbm.at[idx_v.at[slot, j]],
                rows_v.at[slot, pl.ds(j * 128, 128)], gsem.at[slot])

        def scp(slot, c):
            return pltpu.make_async_copy(
                skip_hbm.at[pl.ds(c * chunk, chunk)], io_v.at[slot],
                hsem.at[slot])

        def ocp(slot, c):
            return pltpu.make_async_copy(
                io_v.at[slot], out_hbm.at[pl.ds(c * chunk, chunk)],
                osem.at[slot])

        def issue(slot, i):
            c = base + i
            pltpu.sync_copy(idx_hbm.at[c], idx_v.at[slot])
            for j in range(nidx):
                gcp(slot, j).start()

            @pl.when(i >= 2)
            def _():
                ocp(slot, c - 2).wait()

            scp(slot, c).start()

        def finish(slot, i):
            c = base + i
            for j in range(nidx):
                gcp(slot, j).wait()
            scp(slot, c).wait()

            def e_body(e, inner):
                p = e * MAX_NB
                for g in range(NGRP):
                    sl = pl.ds(g * LANES, LANES)
                    s = _decode_hi(io_v[slot, e, sl])
                    for kk in range(MAX_NB):
                        r = rows_v[slot, p + kk, sl]
                        s = s + (_decode_lo(r) if gather_packed else r)
                    io_v[slot, e, sl] = lax.bitcast_convert_type(
                        jnp.maximum(s, 0.0), jnp.int32)
                return inner

            lax.fori_loop(0, chunk, e_body, 0)
            ocp(slot, c).start()

        issue(0, 0)

        def pair(p, carry):
            i0 = 2 * p
            issue(1, i0 + 1)
            finish(0, i0)

            @pl.when(i0 + 2 < n_my)
            def _():
                issue(0, i0 + 2)

            finish(1, i0 + 1)
            return carry

        lax.fori_loop(0, n_my // 2, pair, 0)

        @pl.when(n_my % 2 == 1)
        def _():
            finish(0, n_my - 1)

        @pl.when(n_my % 2 == 0)
        def _():
            ocp(0, base + n_my - 2).wait()
            ocp(1, base + n_my - 1).wait()

        @pl.when(n_my % 2 == 1)
        def _():
            ocp(1, base + n_my - 2).wait()
            ocp(0, base + n_my - 1).wait()

    return k(hw, lin, idx3)


def _sc_gather_sum(h, idx3, n_out):
    chunk = 64
    nidx = chunk * MAX_NB // 128
    nchunk = n_out // chunk

    @functools.partial(
        pl.kernel, mesh=_mesh(),
        out_type=jax.ShapeDtypeStruct((n_out, HIDDEN), jnp.float32),
        scratch_types=[
            pltpu.VMEM((2, nidx, 128), jnp.int32),
            pltpu.VMEM((2, chunk * MAX_NB, HIDDEN), jnp.float32),
            pltpu.VMEM((2, chunk, HIDDEN), jnp.float32),
            pltpu.SemaphoreType.DMA((2,)),
            pltpu.SemaphoreType.DMA((2,)),
        ],
    )
    def k(h_hbm, idx_hbm, out_hbm, idx_v, rows_v, io_v, gsem, osem):
        base, n_my = _my_range(nchunk)

        def gcp(slot, j):
            return pltpu.make_async_copy(
                h_hbm.at[idx_v.at[slot, j]],
                rows_v.at[slot, pl.ds(j * 128, 128)], gsem.at[slot])

        def ocp(slot, c):
            return pltpu.make_async_copy(
                io_v.at[slot], out_hbm.at[pl.ds(c * chunk, chunk)],
                osem.at[slot])

        def issue(slot, i):
            c = base + i
            pltpu.sync_copy(idx_hbm.at[c], idx_v.at[slot])
            for j in range(nidx):
                gcp(slot, j).start()

        def finish(slot, i):
            c = base + i
            for j in range(nidx):
                gcp(slot, j).wait()

            @pl.when(i >= 2)
            def _():
                ocp(slot, c - 2).wait()

            def e_body(e, inner):
                p = e * MAX_NB
                for g in range(NGRP):
                    sl = pl.ds(g * LANES, LANES)
                    s = rows_v[slot, p, sl]
                    for kk in range(1, MAX_NB):
                        s = s + rows_v[slot, p + kk, sl]
                    io_v[slot, e, sl] = s
                return inner

            lax.fori_loop(0, chunk, e_body, 0)
            ocp(slot, c).start()

        @pl.when(n_my > 0)
        def _():
            issue(0, 0)

        def pair(p, carry):
            i0 = 2 * p
            issue(1, i0 + 1)
            finish(0, i0)

            @pl.when(i0 + 2 < n_my)
            def _():
                issue(0, i0 + 2)

            finish(1, i0 + 1)
            return carry

        lax.fori_loop(0, n_my // 2, pair, 0)

        @pl.when((n_my % 2 == 1) & (n_my > 0))
        def _():
            finish(0, n_my - 1)

        @pl.when((n_my % 2 == 0) & (n_my >= 2))
        def _():
            ocp(0, base + n_my - 2).wait()
            ocp(1, base + n_my - 1).wait()

        @pl.when(n_my % 2 == 1)
        def _():
            ocp(0, base + n_my - 1).wait()

            @pl.when(n_my >= 2)
            def _():
                ocp(1, base + n_my - 2).wait()

    return k(h, idx3)


def _bdot(a, b):
    return jnp.dot(a.astype(jnp.bfloat16), b.astype(jnp.bfloat16),
                   preferred_element_type=jnp.float32)


def _pack_bf16_pair(he, ho):
    bhe = lax.bitcast_convert_type(he, jnp.int32)
    bho = lax.bitcast_convert_type(ho, jnp.int32)
    one = jnp.int32(1)
    half = jnp.int32(0x7FFF)
    re = bhe + half + lax.bitwise_and(lax.shift_right_logical(bhe, 16), one)
    ro = bho + half + lax.bitwise_and(lax.shift_right_logical(bho, 16), one)
    return lax.bitwise_or(lax.shift_right_logical(re, 16),
                          lax.bitwise_and(ro, jnp.int32(-65536)))


def _tc_in(fmess1, bond, w1, w2, wh, b_i, b_h):
    e_rows = fmess1.shape[0]
    be = 1280
    nb = bond.shape[1]

    def body(x_ref, bd_ref, w1_ref, w2_ref, wh_ref, bi_ref, bh_ref, t1_ref):
        h0 = jnp.maximum(
            _bdot(x_ref[...], w1_ref[...])
            + _bdot(bd_ref[...], w2_ref[...])
            + bi_ref[...], 0.0)
        t1_ref[...] = _pack_bf16_pair(_bdot(h0, wh_ref[...]),
                                      h0 + bh_ref[...])

    return pl.pallas_call(
        body,
        grid=(e_rows // be,),
        in_specs=[
            pl.BlockSpec((be, HIDDEN), lambda i: (i, 0)),
            pl.BlockSpec((be, nb), lambda i: (i, 0)),
            pl.BlockSpec((HIDDEN, HIDDEN), lambda i: (0, 0)),
            pl.BlockSpec((nb, HIDDEN), lambda i: (0, 0)),
            pl.BlockSpec((HIDDEN, HIDDEN), lambda i: (0, 0)),
            pl.BlockSpec((1, HIDDEN), lambda i: (0, 0)),
            pl.BlockSpec((1, HIDDEN), lambda i: (0, 0)),
        ],
        out_specs=pl.BlockSpec((be, HIDDEN), lambda i: (i, 0)),
        out_shape=jax.ShapeDtypeStruct((e_rows, HIDDEN), jnp.int32),
    )(fmess1, bond, w1, w2, wh, b_i, b_h)


def _tc_mm(x, w):
    e_rows = x.shape[0]
    be = 1280

    def body(x_ref, w_ref, o_ref):
        o_ref[...] = _bdot(x_ref[...], w_ref[...])

    return pl.pallas_call(
        body,
        grid=(e_rows // be,),
        in_specs=[
            pl.BlockSpec((be, HIDDEN), lambda i: (i, 0)),
            pl.BlockSpec((HIDDEN, HIDDEN), lambda i: (0, 0)),
        ],
        out_specs=pl.BlockSpec((be, HIDDEN), lambda i: (i, 0)),
        out_shape=jax.ShapeDtypeStruct((e_rows, HIDDEN), jnp.float32),
    )(x, w)


def _tc_out(fnode, a, w1, w2, b):
    n_rows = fnode.shape[0]
    bn = 1000

    def body(x_ref, a_ref, w1_ref, w2_ref, b_ref, o_ref):
        o_ref[...] = jnp.maximum(
            _bdot(x_ref[...], w1_ref[...])
            + _bdot(a_ref[...], w2_ref[...])
            + b_ref[...], 0.0)

    return pl.pallas_call(
        body,
        grid=(n_rows // bn,),
        in_specs=[
            pl.BlockSpec((bn, HIDDEN), lambda i: (i, 0)),
            pl.BlockSpec((bn, HIDDEN), lambda i: (i, 0)),
            pl.BlockSpec((HIDDEN, HIDDEN), lambda i: (0, 0)),
            pl.BlockSpec((HIDDEN, HIDDEN), lambda i: (0, 0)),
            pl.BlockSpec((1, HIDDEN), lambda i: (0, 0)),
        ],
        out_specs=pl.BlockSpec((bn, HIDDEN), lambda i: (i, 0)),
        out_shape=jax.ShapeDtypeStruct((n_rows, HIDDEN), jnp.float32),
    )(fnode, a, w1, w2, b)


def kernel(fnode, fmess, agraph, bgraph, W_i, b_i, W_h, b_h, W_o, b_o):
    n_rows, f = fnode.shape
    src2 = fmess[:, 0].astype(jnp.int32).reshape(-1, 1, 128)
    bond = fmess[:, 2:]
    bidx = bgraph.astype(jnp.int32).reshape(-1, 3, 128)
    n_pad = ((n_rows + 63) // 64) * 64
    ag = jnp.concatenate(
        [agraph.astype(jnp.int32),
         jnp.zeros((n_pad - n_rows, MAX_NB), jnp.int32)], axis=0)
    aidx = ag.reshape(-1, 3, 128)

    fmess1 = _sc_gather_rows(fnode, src2)
    t1 = _tc_in(fmess1, bond, W_i[:f], W_i[f:], W_h,
                b_i.reshape(1, HIDDEN), b_h.reshape(1, HIDDEN))
    h = lax.bitcast_convert_type(_sc_msg(t1, t1, bidx, True), jnp.float32)
    hw = _tc_mm(h, W_h)
    h = lax.bitcast_convert_type(_sc_msg(hw, t1, bidx, False), jnp.float32)
    a = _sc_gather_sum(h, aidx, n_pad)[:n_rows]
    return _tc_out(fnode, a, W_o[:f], W_o[f:], b_o.reshape(1, HIDDEN))


# speedup vs baseline: 1.0664x; 1.0664x over previous
import functools

import jax
import jax.numpy as jnp
from jax import lax
from jax.experimental import pallas as pl
from jax.experimental.pallas import tpu as pltpu
from jax.experimental.pallas import tpu_sc as plsc

HIDDEN = 128
MAX_NB = 6
LANES = 16
NW = 32
NGRP = HIDDEN // LANES


def _mesh():
    return plsc.VectorSubcoreMesh(core_axis_name="c", subcore_axis_name="s")


def _worker_id():
    return lax.axis_index("s") * 2 + lax.axis_index("c")


def _my_range(nchunk):
    wid = _worker_id()
    lo = nchunk // NW
    extra = nchunk - lo * NW
    n_my = lo + (wid < extra)
    base = wid * lo + jnp.minimum(wid, extra)
    return base, n_my


def _sc_gather_rows(table, idx3):
    nchunk = idx3.shape[0]
    chunk = 128

    @functools.partial(
        pl.kernel, mesh=_mesh(),
        out_type=jax.ShapeDtypeStruct((nchunk * chunk, HIDDEN), jnp.float32),
        scratch_types=[
            pltpu.VMEM((2, 1, chunk), jnp.int32),
            pltpu.VMEM((2, chunk, HIDDEN), jnp.float32),
            pltpu.SemaphoreType.DMA((2,)),
            pltpu.SemaphoreType.DMA((2,)),
        ],
    )
    def k(table_hbm, idx_hbm, out_hbm, idx_v, rows_v, gsem, osem):
        base, n_my = _my_range(nchunk)

        def gcp(slot):
            return pltpu.make_async_copy(
                table_hbm.at[idx_v.at[slot, 0]],
                rows_v.at[slot], gsem.at[slot])

        def ocp(slot, c):
            return pltpu.make_async_copy(
                rows_v.at[slot], out_hbm.at[pl.ds(c * chunk, chunk)],
                osem.at[slot])

        def issue(slot, i):
            c = base + i
            pltpu.sync_copy(idx_hbm.at[c], idx_v.at[slot])

            @pl.when(i >= 2)
            def _():
                ocp(slot, c - 2).wait()

            gcp(slot).start()

        def finish(slot, i):
            c = base + i
            gcp(slot).wait()
            ocp(slot, c).start()

        issue(0, 0)

        def pair(p, carry):
            i0 = 2 * p
            issue(1, i0 + 1)
            finish(0, i0)

            @pl.when(i0 + 2 < n_my)
            def _():
                issue(0, i0 + 2)

            finish(1, i0 + 1)
            return carry

        lax.fori_loop(0, n_my // 2, pair, 0)

        @pl.when(n_my % 2 == 1)
        def _():
            finish(0, n_my - 1)

        @pl.when(n_my % 2 == 0)
        def _():
            ocp(0, base + n_my - 2).wait()
            ocp(1, base + n_my - 1).wait()

        @pl.when(n_my % 2 == 1)
        def _():
            ocp(0, base + n_my - 1).wait()

            @pl.when(n_my >= 2)
            def _():
                ocp(1, base + n_my - 2).wait()

    return k(table, idx3)


def _sc_msg(hw, skip, idx3):
    e_rows = hw.shape[0]
    chunk = 64
    nidx = chunk * MAX_NB // 128
    nchunk = e_rows // chunk

    @functools.partial(
        pl.kernel, mesh=_mesh(),
        out_type=jax.ShapeDtypeStruct((e_rows, HIDDEN), jnp.float32),
        scratch_types=[
            pltpu.VMEM((2, nidx, 128), jnp.int32),
            pltpu.VMEM((2, chunk * MAX_NB, HIDDEN), jnp.float32),
            pltpu.VMEM((2, chunk, HIDDEN), jnp.float32),
            pltpu.SemaphoreType.DMA((2,)),
            pltpu.SemaphoreType.DMA((2,)),
            pltpu.SemaphoreType.DMA((2,)),
            pltpu.SemaphoreType.DMA((2,)),
        ],
    )
    def k(hw_hbm, skip_hbm, idx_hbm, out_hbm, idx_v, rows_v, io_v, gsem,
          hsem, osem, isem):
        base, n_my = _my_range(nchunk)

        def icp(slot, c):
            return pltpu.make_async_copy(
                idx_hbm.at[c], idx_v.at[slot], isem.at[slot])

        def gcp(slot, j):
            return pltpu.make_async_copy(
                hw_hbm.at[idx_v.at[slot, j]],
                rows_v.at[slot, pl.ds(j * 128, 128)], gsem.at[slot])

        def scp(slot, c):
            return pltpu.make_async_copy(
                skip_hbm.at[pl.ds(c * chunk, chunk)], io_v.at[slot],
                hsem.at[slot])

        def ocp(slot, c):
            return pltpu.make_async_copy(
                io_v.at[slot], out_hbm.at[pl.ds(c * chunk, chunk)],
                osem.at[slot])

        def issue(slot, i):
            c = base + i
            icp(slot, c).wait()
            for j in range(nidx):
                gcp(slot, j).start()

            @pl.when(i >= 2)
            def _():
                ocp(slot, c - 2).wait()

            scp(slot, c).start()

        def finish(slot, i):
            c = base + i
            for j in range(nidx):
                gcp(slot, j).wait()

            @pl.when(i + 2 < n_my)
            def _():
                icp(slot, c + 2).start()

            scp(slot, c).wait()

            def e_body(e, inner):
                p = e * MAX_NB
                for g in range(NGRP):
                    sl = pl.ds(g * LANES, LANES)
                    s = rows_v[slot, p, sl]
                    for kk in range(1, MAX_NB):
                        s = s + rows_v[slot, p + kk, sl]
                    io_v[slot, e, sl] = jnp.maximum(io_v[slot, e, sl] + s,
                                                    0.0)
                return inner

            lax.fori_loop(0, chunk, e_body, 0)
            ocp(slot, c).start()

        icp(0, base).start()
        icp(1, base + 1).start()
        issue(0, 0)

        def pair(p, carry):
            i0 = 2 * p
            issue(1, i0 + 1)
            finish(0, i0)

            @pl.when(i0 + 2 < n_my)
            def _():
                issue(0, i0 + 2)

            finish(1, i0 + 1)
            return carry

        lax.fori_loop(0, n_my // 2, pair, 0)

        @pl.when(n_my % 2 == 1)
        def _():
            finish(0, n_my - 1)

        @pl.when(n_my % 2 == 0)
        def _():
            ocp(0, base + n_my - 2).wait()
            ocp(1, base + n_my - 1).wait()

        @pl.when(n_my % 2 == 1)
        def _():
            ocp(1, base + n_my - 2).wait()
            ocp(0, base + n_my - 1).wait()

    return k(hw, skip, idx3)


def _sc_gather_sum(h, idx3, n_out):
    chunk = 64
    nidx = chunk * MAX_NB // 128
    nchunk = n_out // chunk

    @functools.partial(
        pl.kernel, mesh=_mesh(),
        out_type=jax.ShapeDtypeStruct((n_out, HIDDEN), jnp.float32),
        scratch_types=[
            pltpu.VMEM((2, nidx, 128), jnp.int32),
            pltpu.VMEM((2, chunk * MAX_NB, HIDDEN), jnp.float32),
            pltpu.VMEM((2, chunk, HIDDEN), jnp.float32),
            pltpu.SemaphoreType.DMA((2,)),
            pltpu.SemaphoreType.DMA((2,)),
        ],
    )
    def k(h_hbm, idx_hbm, out_hbm, idx_v, rows_v, io_v, gsem, osem):
        base, n_my = _my_range(nchunk)

        def gcp(slot, j):
            return pltpu.make_async_copy(
                h_hbm.at[idx_v.at[slot, j]],
                rows_v.at[slot, pl.ds(j * 128, 128)], gsem.at[slot])

        def ocp(slot, c):
            return pltpu.make_async_copy(
                io_v.at[slot], out_hbm.at[pl.ds(c * chunk, chunk)],
                osem.at[slot])

        def issue(slot, i):
            c = base + i
            pltpu.sync_copy(idx_hbm.at[c], idx_v.at[slot])
            for j in range(nidx):
                gcp(slot, j).start()

        def finish(slot, i):
            c = base + i
            for j in range(nidx):
                gcp(slot, j).wait()

            @pl.when(i >= 2)
            def _():
                ocp(slot, c - 2).wait()

            def e_body(e, inner):
                p = e * MAX_NB
                for g in range(NGRP):
                    sl = pl.ds(g * LANES, LANES)
                    s = rows_v[slot, p, sl]
                    for kk in range(1, MAX_NB):
                        s = s + rows_v[slot, p + kk, sl]
                    io_v[slot, e, sl] = s
                return inner

            lax.fori_loop(0, chunk, e_body, 0)
            ocp(slot, c).start()

        @pl.when(n_my > 0)
        def _():
            issue(0, 0)

        def pair(p, carry):
            i0 = 2 * p
            issue(1, i0 + 1)
            finish(0, i0)

            @pl.when(i0 + 2 < n_my)
            def _():
                issue(0, i0 + 2)

            finish(1, i0 + 1)
            return carry

        lax.fori_loop(0, n_my // 2, pair, 0)

        @pl.when((n_my % 2 == 1) & (n_my > 0))
        def _():
            finish(0, n_my - 1)

        @pl.when((n_my % 2 == 0) & (n_my >= 2))
        def _():
            ocp(0, base + n_my - 2).wait()
            ocp(1, base + n_my - 1).wait()

        @pl.when(n_my % 2 == 1)
        def _():
            ocp(0, base + n_my - 1).wait()

            @pl.when(n_my >= 2)
            def _():
                ocp(1, base + n_my - 2).wait()

    return k(h, idx3)


def _bdot(a, b):
    return jnp.dot(a.astype(jnp.bfloat16), b.astype(jnp.bfloat16),
                   preferred_element_type=jnp.float32)


def _tc_in(fmess1, bond, w1, w2, wh, b_i, b_h):
    e_rows = fmess1.shape[0]
    be = 1280
    nb = bond.shape[1]

    def body(x_ref, bd_ref, w1_ref, w2_ref, wh_ref, bi_ref, bh_ref, sk_ref,
             hw_ref):
        h0 = jnp.maximum(
            _bdot(x_ref[...], w1_ref[...])
            + _bdot(bd_ref[...], w2_ref[...])
            + bi_ref[...], 0.0)
        sk_ref[...] = h0 + bh_ref[...]
        hw_ref[...] = _bdot(h0, wh_ref[...])

    return pl.pallas_call(
        body,
        grid=(e_rows // be,),
        in_specs=[
            pl.BlockSpec((be, HIDDEN), lambda i: (i, 0)),
            pl.BlockSpec((be, nb), lambda i: (i, 0)),
            pl.BlockSpec((HIDDEN, HIDDEN), lambda i: (0, 0)),
            pl.BlockSpec((nb, HIDDEN), lambda i: (0, 0)),
            pl.BlockSpec((HIDDEN, HIDDEN), lambda i: (0, 0)),
            pl.BlockSpec((1, HIDDEN), lambda i: (0, 0)),
            pl.BlockSpec((1, HIDDEN), lambda i: (0, 0)),
        ],
        out_specs=[pl.BlockSpec((be, HIDDEN), lambda i: (i, 0)),
                   pl.BlockSpec((be, HIDDEN), lambda i: (i, 0))],
        out_shape=[jax.ShapeDtypeStruct((e_rows, HIDDEN), jnp.float32),
                   jax.ShapeDtypeStruct((e_rows, HIDDEN), jnp.float32)],
    )(fmess1, bond, w1, w2, wh, b_i, b_h)


def _tc_mm(x, w):
    e_rows = x.shape[0]
    be = 1280

    def body(x_ref, w_ref, o_ref):
        o_ref[...] = _bdot(x_ref[...], w_ref[...])

    return pl.pallas_call(
        body,
        grid=(e_rows // be,),
        in_specs=[
            pl.BlockSpec((be, HIDDEN), lambda i: (i, 0)),
            pl.BlockSpec((HIDDEN, HIDDEN), lambda i: (0, 0)),
        ],
        out_specs=pl.BlockSpec((be, HIDDEN), lambda i: (i, 0)),
        out_shape=jax.ShapeDtypeStruct((e_rows, HIDDEN), jnp.float32),
    )(x, w)


def _tc_out(fnode, a, w1, w2, b):
    n_rows = fnode.shape[0]
    bn = 1000

    def body(x_ref, a_ref, w1_ref, w2_ref, b_ref, o_ref):
        o_ref[...] = jnp.maximum(
            _bdot(x_ref[...], w1_ref[...])
            + _bdot(a_ref[...], w2_ref[...])
            + b_ref[...], 0.0)

    return pl.pallas_call(
        body,
        grid=(n_rows // bn,),
        in_specs=[
            pl.BlockSpec((bn, HIDDEN), lambda i: (i, 0)),
            pl.BlockSpec((bn, HIDDEN), lambda i: (i, 0)),
            pl.BlockSpec((HIDDEN, HIDDEN), lambda i: (0, 0)),
            pl.BlockSpec((HIDDEN, HIDDEN), lambda i: (0, 0)),
            pl.BlockSpec((1, HIDDEN), lambda i: (0, 0)),
        ],
        out_specs=pl.BlockSpec((bn, HIDDEN), lambda i: (i, 0)),
        out_shape=jax.ShapeDtypeStruct((n_rows, HIDDEN), jnp.float32),
    )(fnode, a, w1, w2, b)


def kernel(fnode, fmess, agraph, bgraph, W_i, b_i, W_h, b_h, W_o, b_o):
    n_rows, f = fnode.shape
    src2 = fmess[:, 0].astype(jnp.int32).reshape(-1, 1, 128)
    bond = fmess[:, 2:]
    bidx = bgraph.astype(jnp.int32).reshape(-1, 3, 128)
    n_pad = ((n_rows + 63) // 64) * 64
    ag = jnp.concatenate(
        [agraph.astype(jnp.int32),
         jnp.zeros((n_pad - n_rows, MAX_NB), jnp.int32)], axis=0)
    aidx = ag.reshape(-1, 3, 128)

    fmess1 = _sc_gather_rows(fnode, src2)
    skip, hw = _tc_in(fmess1, bond, W_i[:f], W_i[f:], W_h,
                      b_i.reshape(1, HIDDEN), b_h.reshape(1, HIDDEN))
    h = _sc_msg(hw, skip, bidx)
    hw = _tc_mm(h, W_h)
    h = _sc_msg(hw, skip, bidx)
    a = _sc_gather_sum(h, aidx, n_pad)[:n_rows]
    return _tc_out(fnode, a, W_o[:f], W_o[f:], b_o.reshape(1, HIDDEN))


# speedup vs baseline: 1.0665x; 1.0001x over previous
import functools

import jax
import jax.numpy as jnp
from jax import lax
from jax.experimental import pallas as pl
from jax.experimental.pallas import tpu as pltpu
from jax.experimental.pallas import tpu_sc as plsc

HIDDEN = 128
MAX_NB = 6
LANES = 16
NW = 32
NGRP = HIDDEN // LANES


def _mesh():
    return plsc.VectorSubcoreMesh(core_axis_name="c", subcore_axis_name="s")


def _worker_id():
    return lax.axis_index("s") * 2 + lax.axis_index("c")


def _my_range(nchunk):
    wid = _worker_id()
    lo = nchunk // NW
    extra = nchunk - lo * NW
    n_my = lo + (wid < extra)
    base = wid * lo + jnp.minimum(wid, extra)
    return base, n_my


def _sc_gather_rows(table, idx3):
    nchunk = idx3.shape[0]
    chunk = 128

    @functools.partial(
        pl.kernel, mesh=_mesh(),
        out_type=jax.ShapeDtypeStruct((nchunk * chunk, HIDDEN), jnp.float32),
        scratch_types=[
            pltpu.VMEM((2, 1, chunk), jnp.int32),
            pltpu.VMEM((2, chunk, HIDDEN), jnp.float32),
            pltpu.SemaphoreType.DMA((2,)),
            pltpu.SemaphoreType.DMA((2,)),
            pltpu.SemaphoreType.DMA((2,)),
        ],
    )
    def k(table_hbm, idx_hbm, out_hbm, idx_v, rows_v, gsem, osem, isem):
        base, n_my = _my_range(nchunk)

        def icp(slot, c):
            return pltpu.make_async_copy(
                idx_hbm.at[c], idx_v.at[slot], isem.at[slot])

        def gcp(slot):
            return pltpu.make_async_copy(
                table_hbm.at[idx_v.at[slot, 0]],
                rows_v.at[slot], gsem.at[slot])

        def ocp(slot, c):
            return pltpu.make_async_copy(
                rows_v.at[slot], out_hbm.at[pl.ds(c * chunk, chunk)],
                osem.at[slot])

        def issue(slot, i):
            c = base + i
            icp(slot, c).wait()

            @pl.when(i >= 2)
            def _():
                ocp(slot, c - 2).wait()

            gcp(slot).start()

        def finish(slot, i):
            c = base + i
            gcp(slot).wait()

            @pl.when(i + 2 < n_my)
            def _():
                icp(slot, c + 2).start()

            ocp(slot, c).start()

        icp(0, base).start()
        icp(1, base + 1).start()
        issue(0, 0)

        def pair(p, carry):
            i0 = 2 * p
            issue(1, i0 + 1)
            finish(0, i0)

            @pl.when(i0 + 2 < n_my)
            def _():
                issue(0, i0 + 2)

            finish(1, i0 + 1)
            return carry

        lax.fori_loop(0, n_my // 2, pair, 0)

        @pl.when(n_my % 2 == 1)
        def _():
            finish(0, n_my - 1)

        @pl.when(n_my % 2 == 0)
        def _():
            ocp(0, base + n_my - 2).wait()
            ocp(1, base + n_my - 1).wait()

        @pl.when(n_my % 2 == 1)
        def _():
            ocp(0, base + n_my - 1).wait()

            @pl.when(n_my >= 2)
            def _():
                ocp(1, base + n_my - 2).wait()

    return k(table, idx3)


def _sc_msg(hw, skip, idx3):
    e_rows = hw.shape[0]
    chunk = 64
    nidx = chunk * MAX_NB // 128
    nchunk = e_rows // chunk

    @functools.partial(
        pl.kernel, mesh=_mesh(),
        out_type=jax.ShapeDtypeStruct((e_rows, HIDDEN), jnp.float32),
        scratch_types=[
            pltpu.VMEM((2, nidx, 128), jnp.int32),
            pltpu.VMEM((2, chunk * MAX_NB, HIDDEN), jnp.float32),
            pltpu.VMEM((2, chunk, HIDDEN), jnp.float32),
            pltpu.SemaphoreType.DMA((2,)),
            pltpu.SemaphoreType.DMA((2,)),
            pltpu.SemaphoreType.DMA((2,)),
            pltpu.SemaphoreType.DMA((2,)),
        ],
    )
    def k(hw_hbm, skip_hbm, idx_hbm, out_hbm, idx_v, rows_v, io_v, gsem,
          hsem, osem, isem):
        base, n_my = _my_range(nchunk)

        def icp(slot, c):
            return pltpu.make_async_copy(
                idx_hbm.at[c], idx_v.at[slot], isem.at[slot])

        def gcp(slot, j):
            return pltpu.make_async_copy(
                hw_hbm.at[idx_v.at[slot, j]],
                rows_v.at[slot, pl.ds(j * 128, 128)], gsem.at[slot])

        def scp(slot, c):
            return pltpu.make_async_copy(
                skip_hbm.at[pl.ds(c * chunk, chunk)], io_v.at[slot],
                hsem.at[slot])

        def ocp(slot, c):
            return pltpu.make_async_copy(
                io_v.at[slot], out_hbm.at[pl.ds(c * chunk, chunk)],
                osem.at[slot])

        def issue(slot, i):
            c = base + i
            icp(slot, c).wait()
            for j in range(nidx):
                gcp(slot, j).start()

            @pl.when(i >= 2)
            def _():
                ocp(slot, c - 2).wait()

            scp(slot, c).start()

        def finish(slot, i):
            c = base + i
            for j in range(nidx):
                gcp(slot, j).wait()

            @pl.when(i + 2 < n_my)
            def _():
                icp(slot, c + 2).start()

            scp(slot, c).wait()

            def e_body(e, inner):
                p = e * MAX_NB
                for g in range(NGRP):
                    sl = pl.ds(g * LANES, LANES)
                    s = rows_v[slot, p, sl]
                    for kk in range(1, MAX_NB):
                        s = s + rows_v[slot, p + kk, sl]
                    io_v[slot, e, sl] = jnp.maximum(io_v[slot, e, sl] + s,
                                                    0.0)
                return inner

            lax.fori_loop(0, chunk, e_body, 0)
            ocp(slot, c).start()

        icp(0, base).start()
        icp(1, base + 1).start()
        issue(0, 0)

        def pair(p, carry):
            i0 = 2 * p
            issue(1, i0 + 1)
            finish(0, i0)

            @pl.when(i0 + 2 < n_my)
            def _():
                issue(0, i0 + 2)

            finish(1, i0 + 1)
            return carry

        lax.fori_loop(0, n_my // 2, pair, 0)

        @pl.when(n_my % 2 == 1)
        def _():
            finish(0, n_my - 1)

        @pl.when(n_my % 2 == 0)
        def _():
            ocp(0, base + n_my - 2).wait()
            ocp(1, base + n_my - 1).wait()

        @pl.when(n_my % 2 == 1)
        def _():
            ocp(1, base + n_my - 2).wait()
            ocp(0, base + n_my - 1).wait()

    return k(hw, skip, idx3)


def _sc_gather_sum(h, idx3, n_out):
    chunk = 64
    nidx = chunk * MAX_NB // 128
    nchunk = n_out // chunk

    @functools.partial(
        pl.kernel, mesh=_mesh(),
        out_type=jax.ShapeDtypeStruct((n_out, HIDDEN), jnp.float32),
        scratch_types=[
            pltpu.VMEM((2, nidx, 128), jnp.int32),
            pltpu.VMEM((2, chunk * MAX_NB, HIDDEN), jnp.float32),
            pltpu.VMEM((2, chunk, HIDDEN), jnp.float32),
            pltpu.SemaphoreType.DMA((2,)),
            pltpu.SemaphoreType.DMA((2,)),
            pltpu.SemaphoreType.DMA((2,)),
        ],
    )
    def k(h_hbm, idx_hbm, out_hbm, idx_v, rows_v, io_v, gsem, osem, isem):
        base, n_my = _my_range(nchunk)

        def icp(slot, c):
            return pltpu.make_async_copy(
                idx_hbm.at[c], idx_v.at[slot], isem.at[slot])

        def gcp(slot, j):
            return pltpu.make_async_copy(
                h_hbm.at[idx_v.at[slot, j]],
                rows_v.at[slot, pl.ds(j * 128, 128)], gsem.at[slot])

        def ocp(slot, c):
            return pltpu.make_async_copy(
                io_v.at[slot], out_hbm.at[pl.ds(c * chunk, chunk)],
                osem.at[slot])

        def issue(slot, i):
            c = base + i
            icp(slot, c).wait()
            for j in range(nidx):
                gcp(slot, j).start()

        def finish(slot, i):
            c = base + i
            for j in range(nidx):
                gcp(slot, j).wait()

            @pl.when(i + 2 < n_my)
            def _():
                icp(slot, c + 2).start()

            @pl.when(i >= 2)
            def _():
                ocp(slot, c - 2).wait()

            def e_body(e, inner):
                p = e * MAX_NB
                for g in range(NGRP):
                    sl = pl.ds(g * LANES, LANES)
                    s = rows_v[slot, p, sl]
                    for kk in range(1, MAX_NB):
                        s = s + rows_v[slot, p + kk, sl]
                    io_v[slot, e, sl] = s
                return inner

            lax.fori_loop(0, chunk, e_body, 0)
            ocp(slot, c).start()

        @pl.when(n_my > 0)
        def _():
            icp(0, base).start()
            icp(1, base + 1).start()
            issue(0, 0)

        def pair(p, carry):
            i0 = 2 * p
            issue(1, i0 + 1)
            finish(0, i0)

            @pl.when(i0 + 2 < n_my)
            def _():
                issue(0, i0 + 2)

            finish(1, i0 + 1)
            return carry

        lax.fori_loop(0, n_my // 2, pair, 0)

        @pl.when((n_my % 2 == 1) & (n_my > 0))
        def _():
            finish(0, n_my - 1)

        @pl.when((n_my % 2 == 0) & (n_my >= 2))
        def _():
            ocp(0, base + n_my - 2).wait()
            ocp(1, base + n_my - 1).wait()

        @pl.when(n_my % 2 == 1)
        def _():
            ocp(0, base + n_my - 1).wait()

            @pl.when(n_my >= 2)
            def _():
                ocp(1, base + n_my - 2).wait()

    return k(h, idx3)


def _bdot(a, b):
    return jnp.dot(a.astype(jnp.bfloat16), b.astype(jnp.bfloat16),
                   preferred_element_type=jnp.float32)


def _tc_in(fmess1, bond, w1, w2, wh, b_i, b_h):
    e_rows = fmess1.shape[0]
    be = 1280
    nb = bond.shape[1]

    def body(x_ref, bd_ref, w1_ref, w2_ref, wh_ref, bi_ref, bh_ref, sk_ref,
             hw_ref):
        h0 = jnp.maximum(
            _bdot(x_ref[...], w1_ref[...])
            + _bdot(bd_ref[...], w2_ref[...])
            + bi_ref[...], 0.0)
        sk_ref[...] = h0 + bh_ref[...]
        hw_ref[...] = _bdot(h0, wh_ref[...])

    return pl.pallas_call(
        body,
        grid=(e_rows // be,),
        in_specs=[
            pl.BlockSpec((be, HIDDEN), lambda i: (i, 0)),
            pl.BlockSpec((be, nb), lambda i: (i, 0)),
            pl.BlockSpec((HIDDEN, HIDDEN), lambda i: (0, 0)),
            pl.BlockSpec((nb, HIDDEN), lambda i: (0, 0)),
            pl.BlockSpec((HIDDEN, HIDDEN), lambda i: (0, 0)),
            pl.BlockSpec((1, HIDDEN), lambda i: (0, 0)),
            pl.BlockSpec((1, HIDDEN), lambda i: (0, 0)),
        ],
        out_specs=[pl.BlockSpec((be, HIDDEN), lambda i: (i, 0)),
                   pl.BlockSpec((be, HIDDEN), lambda i: (i, 0))],
        out_shape=[jax.ShapeDtypeStruct((e_rows, HIDDEN), jnp.float32),
                   jax.ShapeDtypeStruct((e_rows, HIDDEN), jnp.float32)],
    )(fmess1, bond, w1, w2, wh, b_i, b_h)


def _tc_mm(x, w):
    e_rows = x.shape[0]
    be = 1280

    def body(x_ref, w_ref, o_ref):
        o_ref[...] = _bdot(x_ref[...], w_ref[...])

    return pl.pallas_call(
        body,
        grid=(e_rows // be,),
        in_specs=[
            pl.BlockSpec((be, HIDDEN), lambda i: (i, 0)),
            pl.BlockSpec((HIDDEN, HIDDEN), lambda i: (0, 0)),
        ],
        out_specs=pl.BlockSpec((be, HIDDEN), lambda i: (i, 0)),
        out_shape=jax.ShapeDtypeStruct((e_rows, HIDDEN), jnp.float32),
    )(x, w)


def _tc_out(fnode, a, w1, w2, b):
    n_rows = fnode.shape[0]
    bn = 1000

    def body(x_ref, a_ref, w1_ref, w2_ref, b_ref, o_ref):
        o_ref[...] = jnp.maximum(
            _bdot(x_ref[...], w1_ref[...])
            + _bdot(a_ref[...], w2_ref[...])
            + b_ref[...], 0.0)

    return pl.pallas_call(
        body,
        grid=(n_rows // bn,),
        in_specs=[
            pl.BlockSpec((bn, HIDDEN), lambda i: (i, 0)),
            pl.BlockSpec((bn, HIDDEN), lambda i: (i, 0)),
            pl.BlockSpec((HIDDEN, HIDDEN), lambda i: (0, 0)),
            pl.BlockSpec((HIDDEN, HIDDEN), lambda i: (0, 0)),
            pl.BlockSpec((1, HIDDEN), lambda i: (0, 0)),
        ],
        out_specs=pl.BlockSpec((bn, HIDDEN), lambda i: (i, 0)),
        out_shape=jax.ShapeDtypeStruct((n_rows, HIDDEN), jnp.float32),
    )(fnode, a, w1, w2, b)


def kernel(fnode, fmess, agraph, bgraph, W_i, b_i, W_h, b_h, W_o, b_o):
    n_rows, f = fnode.shape
    src2 = fmess[:, 0].astype(jnp.int32).reshape(-1, 1, 128)
    bond = fmess[:, 2:]
    bidx = bgraph.astype(jnp.int32).reshape(-1, 3, 128)
    n_pad = ((n_rows + 63) // 64) * 64
    ag = jnp.concatenate(
        [agraph.astype(jnp.int32),
         jnp.zeros((n_pad - n_rows, MAX_NB), jnp.int32)], axis=0)
    aidx = ag.reshape(-1, 3, 128)

    fmess1 = _sc_gather_rows(fnode, src2)
    skip, hw = _tc_in(fmess1, bond, W_i[:f], W_i[f:], W_h,
                      b_i.reshape(1, HIDDEN), b_h.reshape(1, HIDDEN))
    h = _sc_msg(hw, skip, bidx)
    hw = _tc_mm(h, W_h)
    h = _sc_msg(hw, skip, bidx)
    a = _sc_gather_sum(h, aidx, n_pad)[:n_rows]
    return _tc_out(fnode, a, W_o[:f], W_o[f:], b_o.reshape(1, HIDDEN))
